# baseline (speedup 1.0000x reference)
import functools

import jax
import jax.numpy as jnp
from jax import lax
from jax.experimental import pallas as pl
from jax.experimental.pallas import tpu as pltpu
from jax.experimental.pallas import tpu_sc as plsc

D = 32
S = 50
SP = 56
LP = 128
G = 8


@functools.cache
def _make_gather(R):
    info = plsc.get_sparse_core_info()
    nc, ns = info.num_cores, info.num_subcores
    nw = nc * ns
    r_per_w = R // nw
    n_groups = r_per_w // G
    assert n_groups * G == r_per_w and n_groups % 2 == 0 and n_groups >= 4
    mesh = plsc.VectorSubcoreMesh(core_axis_name="c", subcore_axis_name="s")

    @functools.partial(
        pl.kernel,
        mesh=mesh,
        out_type=jax.ShapeDtypeStruct((R, SP, D), jnp.float32),
        scratch_types=[
            pltpu.VMEM((G, SP), jnp.int32),
            pltpu.VMEM((G, SP), jnp.int32),
            pltpu.VMEM((G, SP, D), jnp.float32),
            pltpu.VMEM((G, SP, D), jnp.float32),
            pltpu.SemaphoreType.DMA,
            pltpu.SemaphoreType.DMA,
            pltpu.SemaphoreType.DMA,
            pltpu.SemaphoreType.DMA,
            pltpu.SemaphoreType.DMA,
            pltpu.SemaphoreType.DMA,
        ],
        compiler_params=pltpu.CompilerParams(use_tc_tiling_on_sc=False),
    )
    def gather_kernel(idx_hbm, table_hbm, out_hbm,
                      idx0, idx1, rows0, rows1,
                      si0, si1, sg0, sg1, so0, so1):
        wid = lax.axis_index("s") * nc + lax.axis_index("c")
        row_base = wid * r_per_w
        idx = (idx0, idx1)
        rows = (rows0, rows1)
        si = (si0, si1)
        sg = (sg0, sg1)
        so = (so0, so1)

        def idx_fetch(g, b):
            pltpu.async_copy(
                idx_hbm.at[pl.ds(row_base + g * G, G), pl.ds(0, SP)],
                idx[b], si[b])

        def idx_wait(b):
            pltpu.make_async_copy(
                idx_hbm.at[pl.ds(0, G), pl.ds(0, SP)], idx[b], si[b]).wait()

        def gather_fire(b):
            for j in range(G):
                pltpu.async_copy(table_hbm.at[idx[b].at[j]],
                                 rows[b].at[j], sg[b])

        def gather_drain(b):
            pltpu.make_async_copy(
                out_hbm.at[pl.ds(0, G)], rows[b], sg[b]).wait()

        def store_fire(g, b):
            pltpu.async_copy(
                rows[b], out_hbm.at[pl.ds(row_base + g * G, G)], so[b])

        def store_drain(b):
            pltpu.make_async_copy(
                rows[b], out_hbm.at[pl.ds(0, G)], so[b]).wait()

        idx_fetch(0, 0)
        idx_fetch(1, 1)
        idx_wait(0)
        gather_fire(0)
        idx_wait(1)
        gather_fire(1)
        gather_drain(0)
        store_fire(0, 0)
        idx_fetch(2, 0)

        def outer(t, carry):
            for b in (0, 1):
                g = 2 * t + b
                bp = 1 - b
                idx_wait(b)
                store_drain(b)
                gather_fire(b)
                gather_drain(bp)
                store_fire(g - 1, bp)

                @pl.when(g + 1 < n_groups)
                def _():
                    idx_fetch(g + 1, bp)
            return carry

        lax.fori_loop(1, n_groups // 2, outer, 0)

        last = n_groups - 1
        gather_drain(last % 2)
        store_fire(last, last % 2)
        store_drain(0)
        store_drain(1)

    return gather_kernel


def kernel(x, weight):
    xi = x.astype(jnp.int32)
    xp = jnp.pad(xi, ((0, 0), (0, LP - xi.shape[1])))
    out = _make_gather(xp.shape[0])(xp, weight)
    return out[:, :S, :]


# speedup vs baseline: 2.8866x; 2.8866x over previous
import functools

import jax
import jax.numpy as jnp
from jax import lax
from jax.experimental import pallas as pl
from jax.experimental.pallas import tpu as pltpu
from jax.experimental.pallas import tpu_sc as plsc

D = 32
S = 50
SP = 56
LP = 128
G = 8


@functools.cache
def _make_gather(R):
    info = plsc.get_sparse_core_info()
    nc, ns = info.num_cores, info.num_subcores
    nw = nc * ns
    r_per_w = R // nw
    n_groups = r_per_w // G
    assert n_groups * G == r_per_w and n_groups % 2 == 0 and n_groups >= 4
    mesh = plsc.VectorSubcoreMesh(core_axis_name="c", subcore_axis_name="s")

    @functools.partial(
        pl.kernel,
        mesh=mesh,
        out_type=jax.ShapeDtypeStruct((R, SP, LP), jnp.float32),
        scratch_types=[
            pltpu.VMEM((G, SP), jnp.int32),
            pltpu.VMEM((G, SP), jnp.int32),
            pltpu.VMEM((G, SP, D), jnp.float32),
            pltpu.VMEM((G, SP, D), jnp.float32),
            pltpu.SemaphoreType.DMA,
            pltpu.SemaphoreType.DMA,
            pltpu.SemaphoreType.DMA,
            pltpu.SemaphoreType.DMA,
            pltpu.SemaphoreType.DMA,
            pltpu.SemaphoreType.DMA,
        ],
        compiler_params=pltpu.CompilerParams(use_tc_tiling_on_sc=False),
    )
    def gather_kernel(idx_hbm, table_hbm, out_hbm,
                      idx0, idx1, rows0, rows1,
                      si0, si1, sg0, sg1, so0, so1):
        wid = lax.axis_index("s") * nc + lax.axis_index("c")
        row_base = wid * r_per_w
        idx = (idx0, idx1)
        rows = (rows0, rows1)
        si = (si0, si1)
        sg = (sg0, sg1)
        so = (so0, so1)

        def idx_fetch(g, b):
            pltpu.async_copy(
                idx_hbm.at[pl.ds(row_base + g * G, G), pl.ds(0, SP)],
                idx[b], si[b])

        def idx_wait(b):
            pltpu.make_async_copy(
                idx_hbm.at[pl.ds(0, G), pl.ds(0, SP)], idx[b], si[b]).wait()

        def gather_fire(b):
            for j in range(G):
                pltpu.async_copy(table_hbm.at[idx[b].at[j]],
                                 rows[b].at[j], sg[b])

        def gather_drain(b):
            pltpu.make_async_copy(
                out_hbm.at[pl.ds(0, G), pl.ds(0, SP), pl.ds(0, D)],
                rows[b], sg[b]).wait()

        def store_fire(g, b):
            pltpu.async_copy(
                rows[b],
                out_hbm.at[pl.ds(row_base + g * G, G), pl.ds(0, SP),
                           pl.ds(0, D)],
                so[b])

        def store_drain(b):
            pltpu.make_async_copy(
                rows[b],
                out_hbm.at[pl.ds(0, G), pl.ds(0, SP), pl.ds(0, D)],
                so[b]).wait()

        idx_fetch(0, 0)
        idx_fetch(1, 1)
        idx_wait(0)
        gather_fire(0)
        idx_wait(1)
        gather_fire(1)
        gather_drain(0)
        store_fire(0, 0)
        idx_fetch(2, 0)

        def outer(t, carry):
            for b in (0, 1):
                g = 2 * t + b
                bp = 1 - b
                idx_wait(b)
                store_drain(b)
                gather_fire(b)
                gather_drain(bp)
                store_fire(g - 1, bp)

                @pl.when(g + 1 < n_groups)
                def _():
                    idx_fetch(g + 1, bp)
            return carry

        lax.fori_loop(1, n_groups // 2, outer, 0)

        last = n_groups - 1
        gather_drain(last % 2)
        store_fire(last, last % 2)
        store_drain(0)
        store_drain(1)

    return gather_kernel


def kernel(x, weight):
    xi = x.astype(jnp.int32)
    n = weight.shape[0]
    fill = (jax.lax.broadcasted_iota(jnp.int32, (xi.shape[0], LP - xi.shape[1]), 0)
            * (LP - xi.shape[1])
            + jax.lax.broadcasted_iota(jnp.int32, (xi.shape[0], LP - xi.shape[1]), 1)) % n
    xp = jnp.concatenate([xi, fill], axis=1)
    out = _make_gather(xp.shape[0])(xp, weight)
    return out[:, :S, :D]
